# Initial kernel scaffold; baseline (speedup 1.0000x reference)
import jax
import jax.numpy as jnp
from jax.experimental import pallas as pl


def kernel(emb_weight, edge_vals, edge_index, drugs):
    raise NotImplementedError("write your pallas kernel here")


# speedup vs baseline: 2.9200x; 2.9200x over previous
import functools

import jax
import jax.numpy as jnp
from jax import lax
from jax.experimental import pallas as pl
from jax.experimental.pallas import tpu as pltpu
from jax.experimental.pallas import tpu_sc as plsc

N = 50000
D = 64
H = 32
E = 800000
B = 4096
NC, NS, L = 2, 16, 16
EPW = 50048
NB = EPW // 128
DPT = B // NS
ZR = 125


def _sc_body(emb2, srcs, dsts, vals, drugs,
             gamma_out,
             xflat,
             acc_sp,
             src_v, dst_v, val_v, rows_v,
             didx_v, drows_v, acc_v, gam_v, zer_v,
             sem):
    c = lax.axis_index("c")
    s = lax.axis_index("s")
    zvec = jnp.zeros((L,), jnp.float32)

    def _zfill(i, carry):
        zer_v[i, 0:L] = zvec
        zer_v[i, L:H] = zvec
        return carry
    lax.fori_loop(0, ZR, _zfill, 0)

    half_base = c * N

    for h in range(DPT // 128):
        pltpu.sync_copy(drugs.at[pl.ds(s * DPT + h * 128, 128)], didx_v)
        for j in range(128 // L):
            didx_v[pl.ds(j * L, L)] = didx_v[pl.ds(j * L, L)] + half_base
        pltpu.async_copy(emb2.at[didx_v], drows_v, sem).wait()

        def _init(e, carry):
            acc_v[h * 128 + e, 0:L] = drows_v[e, 0:L]
            acc_v[h * 128 + e, L:H] = drows_v[e, L:H]
            return carry
        lax.fori_loop(0, 128, _init, 0)

    ebase = s * EPW

    for k in (1, 2, 3, 4):
        for z in range(25):
            pltpu.sync_copy(
                zer_v, acc_sp.at[pl.ds(s * (25 * ZR) + z * ZR, ZR)])
        plsc.subcore_barrier()

        if k == 1:
            src_ref, rd_base = emb2, half_base
        else:
            src_ref = xflat
            rd_base = (((k - 2) % 2) * 2 * N) + half_base

        def _batch(i, carry):
            off = ebase + i * 128
            pltpu.sync_copy(srcs.at[pl.ds(off, 128)], src_v)
            pltpu.sync_copy(dsts.at[pl.ds(off, 128)], dst_v)
            pltpu.sync_copy(vals.at[pl.ds(off, 128)], val_v)
            for j in range(128 // L):
                src_v[pl.ds(j * L, L)] = src_v[pl.ds(j * L, L)] + rd_base
            pltpu.async_copy(src_ref.at[src_v], rows_v, sem).wait()

            def _scale(e, cc):
                v16 = plsc.load_gather(val_v, [jnp.full((L,), e, jnp.int32)])
                rows_v[e, 0:L] = rows_v[e, 0:L] * v16
                rows_v[e, L:H] = rows_v[e, L:H] * v16
                return cc
            lax.fori_loop(0, 128, _scale, 0)
            pltpu.sync_copy(rows_v, acc_sp.at[dst_v], add=True)
            return carry
        lax.fori_loop(0, NB, _batch, 0)
        plsc.subcore_barrier()

        for h in range(DPT // 128):
            pltpu.sync_copy(drugs.at[pl.ds(s * DPT + h * 128, 128)], didx_v)
            pltpu.async_copy(acc_sp.at[didx_v], drows_v, sem).wait()

            def _acc(e, carry):
                acc_v[h * 128 + e, 0:L] = (
                    acc_v[h * 128 + e, 0:L] + drows_v[e, 0:L])
                acc_v[h * 128 + e, L:H] = (
                    acc_v[h * 128 + e, L:H] + drows_v[e, L:H])
                return carry
            lax.fori_loop(0, 128, _acc, 0)

        if k < 4:
            wr_base = (((k - 1) % 2) * 2 * N) + half_base
            pltpu.sync_copy(
                acc_sp.at[pl.ds(s * (25 * ZR), 25 * ZR)],
                xflat.at[pl.ds(wr_base + s * (25 * ZR), 25 * ZR)])
        plsc.subcore_barrier()

    def _gam16(g, carry):
        rows_idx = g * L + lax.iota(jnp.int32, L)

        def _dim(d, ss):
            col = plsc.load_gather(
                acc_v, [rows_idx, jnp.full((L,), d, jnp.int32)])
            return ss + col * col
        ss = lax.fori_loop(0, H, _dim, jnp.zeros((L,), jnp.float32))
        gam_v[pl.ds(g * L, L)] = ss * 0.04
        return carry
    lax.fori_loop(0, DPT // L, _gam16, 0)
    pltpu.sync_copy(gam_v, gamma_out.at[c, pl.ds(s * DPT, DPT)])


@jax.jit
def _run(emb2, srcs, dsts, vals, drugs):
    mesh = plsc.VectorSubcoreMesh(core_axis_name="c", subcore_axis_name="s")
    f = pl.kernel(
        _sc_body,
        out_type=jax.ShapeDtypeStruct((NC, B), jnp.float32),
        mesh=mesh,
        compiler_params=pltpu.CompilerParams(
            needs_layout_passes=False, use_tc_tiling_on_sc=False),
        scratch_types=[
            pltpu.HBM((2 * NC * N, H), jnp.float32),
            pltpu.VMEM_SHARED((N, H), jnp.float32),
            pltpu.VMEM((128,), jnp.int32),
            pltpu.VMEM((128,), jnp.int32),
            pltpu.VMEM((128,), jnp.float32),
            pltpu.VMEM((128, H), jnp.float32),
            pltpu.VMEM((128,), jnp.int32),
            pltpu.VMEM((128, H), jnp.float32),
            pltpu.VMEM((DPT, H), jnp.float32),
            pltpu.VMEM((DPT,), jnp.float32),
            pltpu.VMEM((ZR, H), jnp.float32),
            pltpu.SemaphoreType.DMA,
        ],
    )
    return f(emb2, srcs, dsts, vals, drugs)


def kernel(emb_weight, edge_vals, edge_index, drugs):
    emb2 = (emb_weight.reshape(N, NC, H)
            .transpose(1, 0, 2)
            .reshape(NC * N, H))
    pad = NS * EPW - E
    srcs = jnp.concatenate([edge_index[1], jnp.zeros((pad,), jnp.int32)])
    dsts = jnp.concatenate([edge_index[0], jnp.zeros((pad,), jnp.int32)])
    vals = jnp.concatenate([edge_vals, jnp.zeros((pad,), jnp.float32)])
    parts = _run(emb2, srcs, dsts, vals, drugs)
    return parts[0] + parts[1]
